# Initial kernel scaffold; baseline (speedup 1.0000x reference)
import jax
import jax.numpy as jnp
from jax.experimental import pallas as pl


def kernel(x, ln_g, ln_b, W1, b1, W2, b2):
    raise NotImplementedError("write your pallas kernel here")


# speedup vs baseline: 3.1435x; 3.1435x over previous
import functools

import jax
import jax.numpy as jnp
from jax.experimental import pallas as pl

INPUT_DIM = 256
D_U = 256
KNN_K = 4
B, T = 8, 512
FEAT_NOSEL = INPUT_DIM + 3
FEAT_DIM = FEAT_NOSEL + 1
CTX_DIM = 2 * INPUT_DIM
IN_DIM = FEAT_DIM + CTX_DIM

_HIGHEST = jax.lax.Precision.HIGHEST


def _encoder_kernel(x_ref, ln_g_ref, ln_b_ref, w1_ref, b1_ref, w2_ref, b2_ref,
                    u_ref, sf_ref, sim_ref, ctx_ref):
    xb = x_ref[0]

    sq = jnp.sum(xb * xb, axis=1, keepdims=True)
    gram = jax.lax.dot_general(
        xb, xb, (((1,), (1,)), ((), ())),
        preferred_element_type=jnp.float32, precision=_HIGHEST)
    d2 = jnp.maximum(sq + sq.T - 2.0 * gram, 0.0)
    dist = jnp.sqrt(d2 + 1e-12)
    row = jax.lax.broadcasted_iota(jnp.int32, (T, T), 0)
    col = jax.lax.broadcasted_iota(jnp.int32, (T, T), 1)
    d_ns = jnp.where(row == col, dist + 1e9, dist)

    acc = jnp.zeros((T, 1), jnp.float32)
    d_work = d_ns
    for _ in range(KNN_K):
        m = jnp.min(d_work, axis=1, keepdims=True)
        acc = acc + m
        am = jnp.argmin(d_work, axis=1).astype(jnp.int32)
        d_work = jnp.where(col == am[:, None], 1e9, d_work)
    knn_mean = acc * (1.0 / KNN_K)

    mu_t = jnp.mean(xb, axis=0, keepdims=True)
    diff = xb - mu_t
    cdist = jnp.sqrt(jnp.sum(diff * diff, axis=1, keepdims=True) + 1e-12)
    nrm = jnp.sqrt(sq + 1e-12)
    var_t = jnp.mean(diff * diff, axis=0, keepdims=True)
    sd_t = jnp.sqrt(var_t + 1e-6)
    ctx = jnp.concatenate([mu_t, sd_t], axis=1)
    ctx_ref[0] = ctx

    ones = jnp.ones((T, 1), jnp.float32)
    sf = jnp.concatenate([xb, cdist, knn_mean, nrm, ones], axis=1)
    sf_ref[0] = sf

    ctx_b = jnp.broadcast_to(ctx, (T, CTX_DIM))
    h = jnp.concatenate([sf, ctx_b], axis=1)
    mu_h = jnp.mean(h, axis=1, keepdims=True)
    dh = h - mu_h
    var_h = jnp.mean(dh * dh, axis=1, keepdims=True)
    hn = dh * jax.lax.rsqrt(var_h + 1e-5) * ln_g_ref[0] + ln_b_ref[0]

    h1 = jax.lax.dot_general(
        hn, w1_ref[...], (((1,), (0,)), ((), ())),
        preferred_element_type=jnp.float32, precision=_HIGHEST) + b1_ref[0]
    h1 = 0.5 * h1 * (1.0 + jax.lax.erf(h1 * 0.7071067811865476))
    u = jax.lax.dot_general(
        h1, w2_ref[...], (((1,), (0,)), ((), ())),
        preferred_element_type=jnp.float32, precision=_HIGHEST) + b2_ref[0]
    u_ref[0] = u

    f = sf[:, :FEAT_NOSEL]
    inv = 1.0 / (jnp.sqrt(jnp.sum(f * f, axis=1, keepdims=True)) + 1e-8)
    fn = f * inv
    sim_ref[0] = jax.lax.dot_general(
        fn, fn, (((1,), (1,)), ((), ())),
        preferred_element_type=jnp.float32, precision=_HIGHEST)


@functools.partial(jax.jit, static_argnames=())
def kernel(x, ln_g, ln_b, W1, b1, W2, b2):
    ln_g2 = ln_g.reshape(1, IN_DIM)
    ln_b2 = ln_b.reshape(1, IN_DIM)
    b1_2 = b1.reshape(1, D_U)
    b2_2 = b2.reshape(1, D_U)

    rep = lambda *shape: pl.BlockSpec(shape, lambda b: (0,) * len(shape))
    out_shapes = (
        jax.ShapeDtypeStruct((B, T, D_U), jnp.float32),
        jax.ShapeDtypeStruct((B, T, FEAT_DIM), jnp.float32),
        jax.ShapeDtypeStruct((B, T, T), jnp.float32),
        jax.ShapeDtypeStruct((B, 1, CTX_DIM), jnp.float32),
    )
    u, sf, sim, ctx3 = pl.pallas_call(
        _encoder_kernel,
        grid=(B,),
        in_specs=[
            pl.BlockSpec((1, T, INPUT_DIM), lambda b: (b, 0, 0)),
            rep(1, IN_DIM),
            rep(1, IN_DIM),
            rep(IN_DIM, D_U),
            rep(1, D_U),
            rep(D_U, D_U),
            rep(1, D_U),
        ],
        out_specs=(
            pl.BlockSpec((1, T, D_U), lambda b: (b, 0, 0)),
            pl.BlockSpec((1, T, FEAT_DIM), lambda b: (b, 0, 0)),
            pl.BlockSpec((1, T, T), lambda b: (b, 0, 0)),
            pl.BlockSpec((1, 1, CTX_DIM), lambda b: (b, 0, 0)),
        ),
        out_shape=out_shapes,
    )(x, ln_g2, ln_b2, W1, b1_2, W2, b2_2)
    return (u, sf, sim, ctx3.reshape(B, CTX_DIM))


# speedup vs baseline: 6.1136x; 1.9448x over previous
import functools

import jax
import jax.numpy as jnp
from jax.experimental import pallas as pl

INPUT_DIM = 256
D_U = 256
KNN_K = 4
B, T = 8, 512
FEAT_NOSEL = INPUT_DIM + 3
FEAT_DIM = FEAT_NOSEL + 1
CTX_DIM = 2 * INPUT_DIM
IN_DIM = FEAT_DIM + CTX_DIM

_HIGHEST = jax.lax.Precision.HIGHEST
_INT_INF = 2**31 - 1


def _encoder_kernel(x_ref, w1sf_ref, w1ctx_ref, colsum_ref, cvec_ref, w2_ref,
                    b2_ref, u_ref, sf_ref, sim_ref, ctx_ref):
    xb = x_ref[0]

    sq = jnp.sum(xb * xb, axis=1, keepdims=True)
    gram = jax.lax.dot_general(
        xb, xb, (((1,), (1,)), ((), ())),
        preferred_element_type=jnp.float32, precision=_HIGHEST)
    d2 = jnp.maximum(sq + sq.T - 2.0 * gram, 0.0)
    row = jax.lax.broadcasted_iota(jnp.int32, (T, T), 0)
    col = jax.lax.broadcasted_iota(jnp.int32, (T, T), 1)
    d2_ns = jnp.where(row == col, 1e18, d2)

    key = (jax.lax.bitcast_convert_type(d2_ns, jnp.int32) & (-512)) | col
    acc = jnp.zeros((T, 1), jnp.float32)
    for _ in range(KNN_K):
        kmin = jnp.min(key, axis=1, keepdims=True)
        v2 = jax.lax.bitcast_convert_type(kmin & (-512), jnp.float32)
        acc = acc + jnp.sqrt(v2 + 1e-12)
        key = jnp.where(key == kmin, _INT_INF, key)
    knn_mean = acc * (1.0 / KNN_K)

    mu_t = jnp.mean(xb, axis=0, keepdims=True)
    diff = xb - mu_t
    cdist = jnp.sqrt(jnp.sum(diff * diff, axis=1, keepdims=True) + 1e-12)
    nrm = jnp.sqrt(sq + 1e-12)
    var_t = jnp.mean(diff * diff, axis=0, keepdims=True)
    sd_t = jnp.sqrt(var_t + 1e-6)
    ctx = jnp.concatenate([mu_t, sd_t], axis=1)
    ctx_ref[0] = ctx

    ones = jnp.ones((T, 1), jnp.float32)
    sf = jnp.concatenate([xb, cdist, knn_mean, nrm, ones], axis=1)
    sf_ref[0] = sf

    s_ctx = jnp.sum(ctx, axis=1, keepdims=True)
    s2_ctx = jnp.sum(ctx * ctx, axis=1, keepdims=True)
    mu_h = (jnp.sum(sf, axis=1, keepdims=True) + s_ctx) * (1.0 / IN_DIM)
    ex2 = (jnp.sum(sf * sf, axis=1, keepdims=True) + s2_ctx) * (1.0 / IN_DIM)
    inv_sd = jax.lax.rsqrt(jnp.maximum(ex2 - mu_h * mu_h, 0.0) + 1e-5)

    core = jax.lax.dot_general(
        sf, w1sf_ref[...], (((1,), (0,)), ((), ())),
        preferred_element_type=jnp.float32, precision=_HIGHEST)
    ctxw = jax.lax.dot_general(
        ctx, w1ctx_ref[...], (((1,), (0,)), ((), ())),
        preferred_element_type=jnp.float32, precision=_HIGHEST)
    h1 = inv_sd * (core + ctxw) - (mu_h * inv_sd) * colsum_ref[0] + cvec_ref[0]
    h1 = 0.5 * h1 * (1.0 + jax.lax.erf(h1 * 0.7071067811865476))
    u = jax.lax.dot_general(
        h1, w2_ref[...], (((1,), (0,)), ((), ())),
        preferred_element_type=jnp.float32, precision=_HIGHEST) + b2_ref[0]
    u_ref[0] = u

    rowsq = sq + cdist * cdist + knn_mean * knn_mean + nrm * nrm
    inv = 1.0 / (jnp.sqrt(rowsq) + 1e-8)
    ff = gram + cdist * cdist.T + knn_mean * knn_mean.T + nrm * nrm.T
    sim_ref[0] = (inv * inv.T) * ff


@functools.partial(jax.jit, static_argnames=())
def kernel(x, ln_g, ln_b, W1, b1, W2, b2):
    W1g = ln_g[:, None] * W1
    w1sf = W1g[:FEAT_DIM]
    w1ctx = W1g[FEAT_DIM:]
    colsum = jnp.sum(W1g, axis=0).reshape(1, D_U)
    cvec = (ln_b @ W1 + b1).reshape(1, D_U)
    b2_2 = b2.reshape(1, D_U)

    rep = lambda *shape: pl.BlockSpec(shape, lambda b: (0,) * len(shape))
    out_shapes = (
        jax.ShapeDtypeStruct((B, T, D_U), jnp.float32),
        jax.ShapeDtypeStruct((B, T, FEAT_DIM), jnp.float32),
        jax.ShapeDtypeStruct((B, T, T), jnp.float32),
        jax.ShapeDtypeStruct((B, 1, CTX_DIM), jnp.float32),
    )
    u, sf, sim, ctx3 = pl.pallas_call(
        _encoder_kernel,
        grid=(B,),
        in_specs=[
            pl.BlockSpec((1, T, INPUT_DIM), lambda b: (b, 0, 0)),
            rep(FEAT_DIM, D_U),
            rep(CTX_DIM, D_U),
            rep(1, D_U),
            rep(1, D_U),
            rep(D_U, D_U),
            rep(1, D_U),
        ],
        out_specs=(
            pl.BlockSpec((1, T, D_U), lambda b: (b, 0, 0)),
            pl.BlockSpec((1, T, FEAT_DIM), lambda b: (b, 0, 0)),
            pl.BlockSpec((1, T, T), lambda b: (b, 0, 0)),
            pl.BlockSpec((1, 1, CTX_DIM), lambda b: (b, 0, 0)),
        ),
        out_shape=out_shapes,
    )(x, w1sf, w1ctx, colsum, cvec, W2, b2_2)
    return (u, sf, sim, ctx3.reshape(B, CTX_DIM))
